# Initial kernel scaffold; baseline (speedup 1.0000x reference)
import jax
import jax.numpy as jnp
from jax.experimental import pallas as pl


def kernel(features, edge_index, W, attn_w):
    raise NotImplementedError("write your pallas kernel here")


# speedup vs baseline: 13.4362x; 13.4362x over previous
import dataclasses
import functools

import jax
import jax.numpy as jnp
from jax import lax
from jax.experimental import pallas as pl
from jax.experimental.pallas import tpu as pltpu
from jax.experimental.pallas import tpu_sc as plsc

N = 10000
D = 128
E = 320000
NC, NS = 2, 16
NW = NC * NS
CHUNK = 128
NCHUNK = -(-(E // NW) // CHUNK)
EPT = NCHUNK * CHUNK
E_PAD = EPT * NW
ACC_ROWS = 10240
ROWS_PT = ACC_ROWS // NS
DEN_ROWS = ACC_ROWS // D
ST_ROWS = N + 16


def _tc_project(x, wt, a2):
    def mm(x_ref, w_ref, a_ref, z_ref, st_ref):
        z = lax.dot_general(x_ref[...], w_ref[...], (((1,), (0,)), ((), ())),
                            precision=lax.Precision.HIGHEST,
                            preferred_element_type=jnp.float32)
        z_ref[...] = z
        st_ref[...] = lax.dot_general(z, a_ref[...], (((1,), (0,)), ((), ())),
                                      precision=lax.Precision.HIGHEST,
                                      preferred_element_type=jnp.float32)

    return pl.pallas_call(
        mm,
        out_shape=(jax.ShapeDtypeStruct((N, D), jnp.float32),
                   jax.ShapeDtypeStruct((N, 2), jnp.float32)),
    )(x, wt, a2)


def _sc_aggregate(z, s_pad, t_pad, src_p, dst_p):
    mesh = plsc.VectorSubcoreMesh(core_axis_name="c", subcore_axis_name="s")
    cp = pltpu.CompilerParams()
    if "needs_layout_passes" in pltpu.CompilerParams.__dataclass_fields__:
        cp = dataclasses.replace(cp, needs_layout_passes=False)

    @functools.partial(
        pl.kernel,
        out_type=(jax.ShapeDtypeStruct((NC, ACC_ROWS, D), jnp.float32),
                  jax.ShapeDtypeStruct((NC, DEN_ROWS, D), jnp.float32)),
        mesh=mesh,
        compiler_params=cp,
        scratch_types=[
            pltpu.VMEM((ST_ROWS,), jnp.float32),
            pltpu.VMEM((ST_ROWS,), jnp.float32),
            pltpu.VMEM((CHUNK,), jnp.int32),
            pltpu.VMEM((CHUNK,), jnp.int32),
            pltpu.VMEM((CHUNK, D), jnp.float32),
            pltpu.VMEM((CHUNK,), jnp.float32),
            pltpu.VMEM((DEN_ROWS, D), jnp.float32),
            pltpu.VMEM((DEN_ROWS,), jnp.int32),
            pltpu.VMEM_SHARED((ACC_ROWS, D), jnp.float32),
            pltpu.VMEM_SHARED((DEN_ROWS, D), jnp.float32),
        ],
    )
    def k(z_hbm, s_hbm, t_hbm, src_hbm, dst_hbm, num_hbm, den_hbm,
          s_v, t_v, idx_s, idx_d, zrow, wbuf, den_v, den_idx, acc, den_sh):
        cid = lax.axis_index("c")
        sid = lax.axis_index("s")
        wid = cid * NS + sid
        base = wid * EPT
        zv = jnp.zeros((16,), jnp.float32)
        lane = jnp.arange(16, dtype=jnp.int32)

        pltpu.sync_copy(s_hbm, s_v)
        pltpu.sync_copy(t_hbm, t_v)

        @pl.loop(0, DEN_ROWS)
        def _(r):
            for kk in range(D // 16):
                den_v[r, pl.ds(kk * 16, 16)] = zv

        for g in range(DEN_ROWS // 16):
            den_idx[pl.ds(g * 16, 16)] = g * 16 + lane

        @pl.loop(0, CHUNK)
        def _(r):
            for kk in range(D // 16):
                zrow[r, pl.ds(kk * 16, 16)] = zv

        for j in range(ROWS_PT // CHUNK):
            pltpu.sync_copy(zrow, acc.at[pl.ds(sid * ROWS_PT + j * CHUNK, CHUNK)])

        @pl.when(sid == 0)
        def _():
            pltpu.sync_copy(zrow.at[pl.ds(0, DEN_ROWS)], den_sh)

        plsc.subcore_barrier()

        @pl.loop(0, NCHUNK)
        def _(ci):
            off = base + ci * CHUNK
            pltpu.sync_copy(src_hbm.at[pl.ds(off, CHUNK)], idx_s)
            pltpu.sync_copy(dst_hbm.at[pl.ds(off, CHUNK)], idx_d)
            pltpu.sync_copy(z_hbm.at[idx_s], zrow)
            for g in range(CHUNK // 16):
                si = idx_s[pl.ds(g * 16, 16)]
                di = idx_d[pl.ds(g * 16, 16)]
                x = plsc.load_gather(s_v, [si]) + plsc.load_gather(t_v, [di])
                x = jnp.where(x >= 0.0, x, x * jnp.float32(0.01))
                w = jnp.exp(x)
                wbuf[pl.ds(g * 16, 16)] = w
                plsc.addupdate_scatter(
                    den_v, [lax.shift_right_logical(di, 7),
                            lax.bitwise_and(di, jnp.int32(D - 1))], w)

            @pl.loop(0, CHUNK)
            def _(r):
                wv = plsc.load_gather(wbuf, [jnp.full((16,), r, jnp.int32)])
                for kk in range(D // 16):
                    zrow[r, pl.ds(kk * 16, 16)] = zrow[r, pl.ds(kk * 16, 16)] * wv

            pltpu.sync_copy(zrow, acc.at[idx_d], add=True)

        pltpu.sync_copy(den_v, den_sh.at[den_idx], add=True)
        plsc.subcore_barrier()

        pltpu.sync_copy(acc.at[pl.ds(sid * ROWS_PT, ROWS_PT)],
                        num_hbm.at[cid, pl.ds(sid * ROWS_PT, ROWS_PT)])

        @pl.when(sid == 0)
        def _():
            pltpu.sync_copy(den_sh, den_hbm.at[cid])

    return k(z, s_pad, t_pad, src_p, dst_p)


def _tc_combine(num, den):
    def comb(p_ref, d_ref, o_ref):
        hn = p_ref[0, :, :] + p_ref[1, :, :]
        dn = d_ref[0, :, :] + d_ref[1, :, :]
        h = jnp.where(dn > 0.0, hn / dn, 0.0)
        o_ref[...] = h[:N, :]

    return pl.pallas_call(
        comb,
        out_shape=jax.ShapeDtypeStruct((N, D), jnp.float32),
    )(num, den)


def kernel(features, edge_index, W, attn_w):
    wt = W.T
    a2 = jnp.stack([attn_w[0, :D], attn_w[0, D:]], axis=1)
    z, st = _tc_project(features, wt, a2)
    s_pad = jnp.concatenate([st[:, 0], jnp.zeros((ST_ROWS - N,), jnp.float32)])
    t_pad = jnp.concatenate([st[:, 1], jnp.zeros((ST_ROWS - N,), jnp.float32)])
    src = edge_index[0].astype(jnp.int32)
    dst = edge_index[1].astype(jnp.int32)
    pad = E_PAD - E
    src_p = jnp.concatenate([src, jnp.zeros((pad,), jnp.int32)])
    dst_p = jnp.concatenate([dst, jnp.full((pad,), N, jnp.int32)])
    num, den = _sc_aggregate(z, s_pad, t_pad, src_p, dst_p)
    return _tc_combine(num, den.reshape(NC, ACC_ROWS, 1))
